# baseline (speedup 1.0000x reference)
import jax
import jax.numpy as jnp
from jax.experimental import pallas as pl
from jax.experimental.pallas import tpu as pltpu

_B, _MD, _CD, _H, _E = 4096, 138, 11, 512, 8
_ID = _MD + _CD
_BB = 512
_EH = _E * _H


def _elu(v):
    return jnp.where(v > 0, v, jnp.exp(jnp.minimum(v, 0.0)) - 1.0)


def _moe_body(motion_ref, command_ref, gW1_ref, gb1_ref, gW2_ref, gb2_ref,
              W0_ref, b0_ref, W1_ref, b1_ref, W2_ref, b2_ref,
              W3_ref, b3_ref, out_ref,
              g1s, g1b, g2s, g2b, W0s, b0s, Wc1, b1s, Wc2, b2s, Wc3, b3s, rs):
    f32 = jnp.float32
    bf = jnp.bfloat16

    g1s[...] = gW1_ref[...].T.astype(bf)
    g1b[...] = gb1_ref[...].T
    g2s[...] = gW2_ref[...].T.astype(bf)
    g2b[...] = gb2_ref[...].T
    W0s[...] = W0_ref[...].astype(bf)
    b0s[...] = b0_ref[...].T.astype(bf)
    b1s[...] = b1_ref[...].T.astype(bf)
    b2s[...] = b2_ref[...].T.astype(bf)
    b3s[...] = b3_ref[...].T.astype(bf)
    for e in range(_E):
        Wc1[:, e * _H:(e + 1) * _H] = W1_ref[e].astype(bf)
        Wc2[:, e * _H:(e + 1) * _H] = W2_ref[e].astype(bf)
        Wc3[:, e * _H:(e + 1) * _H] = W3_ref[e].astype(bf)

    def chunk(j, carry):
        sl = pl.ds(j * _BB, _BB)
        xt = jnp.concatenate([motion_ref[sl, :].T, command_ref[sl, :].T],
                             axis=0).astype(bf)

        h = jnp.dot(g1s[...], xt, preferred_element_type=f32) + g1b[...]
        h = _elu(h)
        logits = (jnp.dot(g2s[...], h.astype(bf), preferred_element_type=f32)
                  + g2b[...])
        mx = jnp.max(logits, axis=0, keepdims=True)
        p = jnp.exp(logits - mx)
        coeffs = p / jnp.sum(p, axis=0, keepdims=True)
        cb = coeffs.astype(bf)

        acc = jnp.dot(b0s[...], cb, preferred_element_type=f32)
        for e in range(_E):
            me = jnp.dot(W0s[e], xt, preferred_element_type=f32)
            acc = acc + coeffs[e:e + 1, :] * me
        y = _elu(acc)

        def layer(inp_f32, Wc, bs, act):
            inp_bf = inp_f32.astype(bf)
            for e in range(_E):
                rs[e * _H:(e + 1) * _H, :] = inp_bf * cb[e:e + 1, :]
            acc = jnp.dot(Wc[...], rs[...], preferred_element_type=f32)
            acc = acc + jnp.dot(bs[...], cb, preferred_element_type=f32)
            return _elu(acc) if act else acc

        y = layer(y, Wc1, b1s, True)
        y = layer(y, Wc2, b2s, True)
        y = layer(y, Wc3, b3s, False)
        out_ref[sl, :] = y.T
        return carry

    jax.lax.fori_loop(0, _B // _BB, chunk, 0, unroll=2)


def kernel(motion, command, gW1, gb1, gW2, gb2, W0, b0, W1, b1, W2, b2, W3, b3):
    bf = jnp.bfloat16
    f32 = jnp.float32
    scratch_shapes = [
        pltpu.VMEM((_H, _ID), bf),
        pltpu.VMEM((_H, 1), f32),
        pltpu.VMEM((_E, _H), bf),
        pltpu.VMEM((_E, 1), f32),
        pltpu.VMEM((_E, _H, _ID), bf),
        pltpu.VMEM((_H, _E), bf),
        pltpu.VMEM((_H, _EH), bf),
        pltpu.VMEM((_H, _E), bf),
        pltpu.VMEM((_H, _EH), bf),
        pltpu.VMEM((_H, _E), bf),
        pltpu.VMEM((_MD, _EH), bf),
        pltpu.VMEM((_MD, _E), bf),
        pltpu.VMEM((_EH, _BB), bf),
    ]
    out = pl.pallas_call(
        _moe_body,
        out_shape=jax.ShapeDtypeStruct((_B, _MD), jnp.float32),
        scratch_shapes=scratch_shapes,
    )(motion, command, gW1, gb1.reshape(1, _H), gW2, gb2.reshape(1, _E),
      W0, b0, W1, b1, W2, b2, W3, b3)
    return out


# speedup vs baseline: 1.0827x; 1.0827x over previous
import jax
import jax.numpy as jnp
from jax.experimental import pallas as pl
from jax.experimental.pallas import tpu as pltpu

_B, _MD, _CD, _H, _E = 4096, 138, 11, 512, 8
_ID = _MD + _CD
_IDP = 160
_BB = 1024
_EH = _E * _H
_EIDP = _E * _IDP


def _elu_bf(v):
    return jnp.where(v > 0, v, jnp.exp(jnp.minimum(v, 0.0)) - 1.0
                     ).astype(jnp.bfloat16)


def _moe_body(motion_ref, command_ref, gW1_ref, gb1_ref, gW2_ref, gb2_ref,
              W0_ref, b0_ref, W1_ref, b1_ref, W2_ref, b2_ref,
              W3_ref, b3_ref, out_ref,
              g1s, g1b, g2s, g2b, Wc0, b0s, Wc1, b1s, Wc2, b2s, Wc3, b3s,
              rs0, rs):
    f32 = jnp.float32
    bf = jnp.bfloat16

    g1s[...] = gW1_ref[...].T.astype(bf)
    g1b[...] = gb1_ref[...].T
    g2s[...] = gW2_ref[...].T.astype(bf)
    g2b[...] = gb2_ref[...].T
    b0s[...] = b0_ref[...].T.astype(bf)
    b1s[...] = b1_ref[...].T.astype(bf)
    b2s[...] = b2_ref[...].T.astype(bf)
    b3s[...] = b3_ref[...].T.astype(bf)
    Wc0[...] = jnp.zeros((_H, _EIDP), bf)
    rs0[...] = jnp.zeros((_EIDP, _BB), bf)
    for e in range(_E):
        Wc0[:, e * _IDP:e * _IDP + _ID] = W0_ref[e].astype(bf)
        Wc1[:, e * _H:(e + 1) * _H] = W1_ref[e].astype(bf)
        Wc2[:, e * _H:(e + 1) * _H] = W2_ref[e].astype(bf)
        Wc3[:, e * _H:(e + 1) * _H] = W3_ref[e].astype(bf)

    def chunk(j, carry):
        sl = pl.ds(j * _BB, _BB)
        xt = jnp.concatenate([motion_ref[sl, :].T, command_ref[sl, :].T],
                             axis=0).astype(bf)

        h = jnp.dot(g1s[...], xt, preferred_element_type=f32) + g1b[...]
        h = _elu_bf(h)
        logits = (jnp.dot(g2s[...], h, preferred_element_type=f32)
                  + g2b[...])
        mx = jnp.max(logits, axis=0, keepdims=True)
        p = jnp.exp(logits - mx)
        coeffs = p / jnp.sum(p, axis=0, keepdims=True)
        cb = coeffs.astype(bf)

        for e in range(_E):
            rs0[e * _IDP:e * _IDP + _ID, :] = xt * cb[e:e + 1, :]
        acc = jnp.dot(Wc0[...], rs0[...], preferred_element_type=f32)
        acc = acc + jnp.dot(b0s[...], cb, preferred_element_type=f32)
        y = _elu_bf(acc)

        def layer(inp_bf, Wc, bs, act):
            for e in range(_E):
                rs[e * _H:(e + 1) * _H, :] = inp_bf * cb[e:e + 1, :]
            acc = jnp.dot(Wc[...], rs[...], preferred_element_type=f32)
            acc = acc + jnp.dot(bs[...], cb, preferred_element_type=f32)
            return _elu_bf(acc) if act else acc

        y = layer(y, Wc1, b1s, True)
        y = layer(y, Wc2, b2s, True)
        y = layer(y, Wc3, b3s, False)
        out_ref[sl, :] = y.T
        return carry

    jax.lax.fori_loop(0, _B // _BB, chunk, 0, unroll=2)


def kernel(motion, command, gW1, gb1, gW2, gb2, W0, b0, W1, b1, W2, b2, W3, b3):
    bf = jnp.bfloat16
    f32 = jnp.float32
    scratch_shapes = [
        pltpu.VMEM((_H, _ID), bf),
        pltpu.VMEM((_H, 1), f32),
        pltpu.VMEM((_E, _H), bf),
        pltpu.VMEM((_E, 1), f32),
        pltpu.VMEM((_H, _EIDP), bf),
        pltpu.VMEM((_H, _E), bf),
        pltpu.VMEM((_H, _EH), bf),
        pltpu.VMEM((_H, _E), bf),
        pltpu.VMEM((_H, _EH), bf),
        pltpu.VMEM((_H, _E), bf),
        pltpu.VMEM((_MD, _EH), bf),
        pltpu.VMEM((_MD, _E), bf),
        pltpu.VMEM((_EIDP, _BB), bf),
        pltpu.VMEM((_EH, _BB), bf),
    ]
    out = pl.pallas_call(
        _moe_body,
        out_shape=jax.ShapeDtypeStruct((_B, _MD), jnp.float32),
        scratch_shapes=scratch_shapes,
    )(motion, command, gW1, gb1.reshape(1, _H), gW2, gb2.reshape(1, _E),
      W0, b0, W1, b1, W2, b2, W3, b3)
    return out
